# baseline (speedup 1.0000x reference)
import functools

import jax
import jax.numpy as jnp
import numpy as np
from jax import lax
from jax.experimental import pallas as pl
from jax.experimental.pallas import tpu as pltpu
from jax.experimental.pallas import tpu_sc as plsc

T_STEPS = 1000
BETA_START = 0.0001
BETA_END = 0.02
TAB_PAD = 1024

B = 4096
ROW = 200 * 64

NUM_WORKERS = 32
B_PER_W = B // NUM_WORKERS


TAB_W = 128


def _schedule_table():
    betas = np.linspace(BETA_START, BETA_END, T_STEPS).astype(np.float32)
    alphas = (np.float32(1.0) - betas).astype(np.float32)
    abars = np.cumprod(alphas, dtype=np.float32)
    abars = np.concatenate([np.ones((1,), np.float32), abars])
    sqrt_ab = np.sqrt(abars).astype(np.float32)
    sqrt_1m = np.sqrt(np.maximum(np.float32(1.0) - abars, np.float32(0.0)))
    sqrt_1m = sqrt_1m.astype(np.float32)
    tab = np.zeros((TAB_PAD, TAB_W), np.float32)
    tab[: abars.shape[0], :8] = sqrt_ab[:, None]
    tab[: abars.shape[0], 8:] = sqrt_1m[:, None]
    return tab


_SCHED_TAB = _schedule_table()


def _sc_gather_coeffs(t):
    mesh = plsc.VectorSubcoreMesh(core_axis_name="c", subcore_axis_name="s")

    @functools.partial(
        pl.kernel,
        mesh=mesh,
        out_type=jax.ShapeDtypeStruct((B, TAB_W), jnp.float32),
        scratch_types=[
            pltpu.VMEM((B_PER_W,), jnp.int32),
            pltpu.VMEM((B_PER_W, TAB_W), jnp.float32),
            pltpu.SemaphoreType.DMA,
        ],
    )
    def k(tab_hbm, t_hbm, out_hbm, vidx, vrows, sem):
        wid = lax.axis_index("s") * 2 + lax.axis_index("c")
        base = wid * B_PER_W
        pltpu.sync_copy(t_hbm.at[pl.ds(base, B_PER_W)], vidx)
        pltpu.async_copy(tab_hbm.at[vidx], vrows, sem).wait()
        pltpu.sync_copy(vrows, out_hbm.at[pl.ds(base, B_PER_W)])

    return k(jnp.asarray(_SCHED_TAB), t)


BT = 16


def _tc_body(co_ref, cd_ref, nz_ref, out_ref):
    ab = co_ref[:, 0:1].reshape(BT, 1, 1)
    m1 = co_ref[:, 8:9].reshape(BT, 1, 1)
    out_ref[...] = ab * cd_ref[...] + m1 * nz_ref[...]


def kernel(clean_delta, t, noise):
    t = t.astype(jnp.int32)
    coeffs = _sc_gather_coeffs(t)

    out = pl.pallas_call(
        _tc_body,
        grid=(B // BT,),
        in_specs=[
            pl.BlockSpec((BT, TAB_W), lambda i: (i, 0)),
            pl.BlockSpec((BT, 200, 64), lambda i: (i, 0, 0)),
            pl.BlockSpec((BT, 200, 64), lambda i: (i, 0, 0)),
        ],
        out_specs=pl.BlockSpec((BT, 200, 64), lambda i: (i, 0, 0)),
        out_shape=jax.ShapeDtypeStruct((B, 200, 64), jnp.float32),
        compiler_params=pltpu.CompilerParams(
            dimension_semantics=("arbitrary",),
        ),
    )(coeffs, clean_delta, noise)
    return out


# speedup vs baseline: 4.8432x; 4.8432x over previous
import functools

import jax
import jax.numpy as jnp
import numpy as np
from jax import lax
from jax.experimental import pallas as pl
from jax.experimental.pallas import tpu as pltpu
from jax.experimental.pallas import tpu_sc as plsc

T_STEPS = 1000
BETA_START = 0.0001
BETA_END = 0.02
TAB_PAD = 1024

B = 4096
ROW = 200 * 64

NUM_WORKERS = 32
B_PER_W = B // NUM_WORKERS


TAB_W = 128


def _schedule_table():
    betas = np.linspace(BETA_START, BETA_END, T_STEPS).astype(np.float32)
    alphas = (np.float32(1.0) - betas).astype(np.float32)
    abars = np.cumprod(alphas, dtype=np.float32)
    abars = np.concatenate([np.ones((1,), np.float32), abars])
    sqrt_ab = np.sqrt(abars).astype(np.float32)
    sqrt_1m = np.sqrt(np.maximum(np.float32(1.0) - abars, np.float32(0.0)))
    sqrt_1m = sqrt_1m.astype(np.float32)
    tab = np.zeros((TAB_PAD, TAB_W), np.float32)
    tab[: abars.shape[0], :8] = sqrt_ab[:, None]
    tab[: abars.shape[0], 8:] = sqrt_1m[:, None]
    return tab


_SCHED_TAB = _schedule_table()


def _sc_gather_coeffs(t):
    mesh = plsc.VectorSubcoreMesh(core_axis_name="c", subcore_axis_name="s")

    @functools.partial(
        pl.kernel,
        mesh=mesh,
        out_type=jax.ShapeDtypeStruct((B, TAB_W), jnp.float32),
        scratch_types=[
            pltpu.VMEM((B_PER_W,), jnp.int32),
            pltpu.VMEM((B_PER_W, TAB_W), jnp.float32),
            pltpu.SemaphoreType.DMA,
        ],
    )
    def k(tab_hbm, t_hbm, out_hbm, vidx, vrows, sem):
        wid = lax.axis_index("s") * 2 + lax.axis_index("c")
        base = wid * B_PER_W
        pltpu.sync_copy(t_hbm.at[pl.ds(base, B_PER_W)], vidx)
        pltpu.async_copy(tab_hbm.at[vidx], vrows, sem).wait()
        pltpu.sync_copy(vrows, out_hbm.at[pl.ds(base, B_PER_W)])

    return k(jnp.asarray(_SCHED_TAB), t)


RS = 64


def _tc_body(ab_ref, m1_ref, cd_ref, nz_ref, out_ref):
    out_ref[...] = ab_ref[...] * cd_ref[...] + m1_ref[...] * nz_ref[...]


def kernel(clean_delta, t, noise):
    t = t.astype(jnp.int32)
    coeffs = _sc_gather_coeffs(t)
    ab = coeffs[:, 0].reshape(1, B)
    m1 = coeffs[:, 8].reshape(1, B)

    cd2 = clean_delta.transpose(1, 2, 0).reshape(ROW, B)
    nz2 = noise.transpose(1, 2, 0).reshape(ROW, B)

    out = pl.pallas_call(
        _tc_body,
        grid=(ROW // RS,),
        in_specs=[
            pl.BlockSpec((1, B), lambda i: (0, 0)),
            pl.BlockSpec((1, B), lambda i: (0, 0)),
            pl.BlockSpec((RS, B), lambda i: (i, 0)),
            pl.BlockSpec((RS, B), lambda i: (i, 0)),
        ],
        out_specs=pl.BlockSpec((RS, B), lambda i: (i, 0)),
        out_shape=jax.ShapeDtypeStruct((ROW, B), jnp.float32),
        compiler_params=pltpu.CompilerParams(
            dimension_semantics=("arbitrary",),
        ),
    )(ab, m1, cd2, nz2)
    return out.reshape(200, 64, B).transpose(2, 0, 1)


# speedup vs baseline: 5.6557x; 1.1678x over previous
import functools

import jax
import jax.numpy as jnp
import numpy as np
from jax import lax
from jax.experimental import pallas as pl
from jax.experimental.pallas import tpu as pltpu
from jax.experimental.pallas import tpu_sc as plsc

T_STEPS = 1000
BETA_START = 0.0001
BETA_END = 0.02
TAB_PAD = 1024

B = 4096
ROW = 200 * 64

NUM_WORKERS = 32
B_PER_W = B // NUM_WORKERS


TAB_W = 128


def _schedule_table():
    betas = np.linspace(BETA_START, BETA_END, T_STEPS).astype(np.float32)
    alphas = (np.float32(1.0) - betas).astype(np.float32)
    abars = np.cumprod(alphas, dtype=np.float32)
    abars = np.concatenate([np.ones((1,), np.float32), abars])
    sqrt_ab = np.sqrt(abars).astype(np.float32)
    sqrt_1m = np.sqrt(np.maximum(np.float32(1.0) - abars, np.float32(0.0)))
    sqrt_1m = sqrt_1m.astype(np.float32)
    tab = np.zeros((TAB_PAD, TAB_W), np.float32)
    tab[: abars.shape[0], :8] = sqrt_ab[:, None]
    tab[: abars.shape[0], 8:] = sqrt_1m[:, None]
    return tab


_SCHED_TAB = _schedule_table()


def _sc_gather_coeffs(t):
    mesh = plsc.VectorSubcoreMesh(core_axis_name="c", subcore_axis_name="s")

    @functools.partial(
        pl.kernel,
        mesh=mesh,
        out_type=jax.ShapeDtypeStruct((B, TAB_W), jnp.float32),
        scratch_types=[
            pltpu.VMEM((B_PER_W,), jnp.int32),
            pltpu.VMEM((B_PER_W, TAB_W), jnp.float32),
            pltpu.SemaphoreType.DMA,
        ],
    )
    def k(tab_hbm, t_hbm, out_hbm, vidx, vrows, sem):
        wid = lax.axis_index("s") * 2 + lax.axis_index("c")
        base = wid * B_PER_W
        pltpu.sync_copy(t_hbm.at[pl.ds(base, B_PER_W)], vidx)
        pltpu.async_copy(tab_hbm.at[vidx], vrows, sem).wait()
        pltpu.sync_copy(vrows, out_hbm.at[pl.ds(base, B_PER_W)])

    return k(jnp.asarray(_SCHED_TAB), t)


RS = 128


def _tc_body(ab_ref, m1_ref, cd_ref, nz_ref, out_ref):
    out_ref[...] = ab_ref[...] * cd_ref[...] + m1_ref[...] * nz_ref[...]


def kernel(clean_delta, t, noise):
    t = t.astype(jnp.int32)
    coeffs = _sc_gather_coeffs(t)
    ab = coeffs[:, 0].reshape(1, B)
    m1 = coeffs[:, 8].reshape(1, B)

    cd2 = clean_delta.transpose(1, 2, 0).reshape(ROW, B)
    nz2 = noise.transpose(1, 2, 0).reshape(ROW, B)

    out = pl.pallas_call(
        _tc_body,
        grid=(ROW // RS,),
        in_specs=[
            pl.BlockSpec((1, B), lambda i: (0, 0)),
            pl.BlockSpec((1, B), lambda i: (0, 0)),
            pl.BlockSpec((RS, B), lambda i: (i, 0)),
            pl.BlockSpec((RS, B), lambda i: (i, 0)),
        ],
        out_specs=pl.BlockSpec((RS, B), lambda i: (i, 0)),
        out_shape=jax.ShapeDtypeStruct((ROW, B), jnp.float32),
        compiler_params=pltpu.CompilerParams(
            dimension_semantics=("arbitrary",),
        ),
    )(ab, m1, cd2, nz2)
    return out.reshape(200, 64, B).transpose(2, 0, 1)


# speedup vs baseline: 5.8515x; 1.0346x over previous
import functools

import jax
import jax.numpy as jnp
import numpy as np
from jax import lax
from jax.experimental import pallas as pl
from jax.experimental.pallas import tpu as pltpu
from jax.experimental.pallas import tpu_sc as plsc

T_STEPS = 1000
BETA_START = 0.0001
BETA_END = 0.02
TAB_PAD = 1024

B = 4096
ROW = 200 * 64

NUM_WORKERS = 32
B_PER_W = B // NUM_WORKERS


TAB_W = 128


def _schedule_table():
    betas = np.linspace(BETA_START, BETA_END, T_STEPS).astype(np.float32)
    alphas = (np.float32(1.0) - betas).astype(np.float32)
    abars = np.cumprod(alphas, dtype=np.float32)
    abars = np.concatenate([np.ones((1,), np.float32), abars])
    sqrt_ab = np.sqrt(abars).astype(np.float32)
    sqrt_1m = np.sqrt(np.maximum(np.float32(1.0) - abars, np.float32(0.0)))
    sqrt_1m = sqrt_1m.astype(np.float32)
    tab = np.zeros((TAB_PAD, TAB_W), np.float32)
    tab[: abars.shape[0], :8] = sqrt_ab[:, None]
    tab[: abars.shape[0], 8:] = sqrt_1m[:, None]
    return tab


_SCHED_TAB = _schedule_table()


def _sc_gather_coeffs(t):
    mesh = plsc.VectorSubcoreMesh(core_axis_name="c", subcore_axis_name="s")

    @functools.partial(
        pl.kernel,
        mesh=mesh,
        out_type=jax.ShapeDtypeStruct((B, TAB_W), jnp.float32),
        scratch_types=[
            pltpu.VMEM((B_PER_W,), jnp.int32),
            pltpu.VMEM((B_PER_W, TAB_W), jnp.float32),
            pltpu.SemaphoreType.DMA,
        ],
    )
    def k(tab_hbm, t_hbm, out_hbm, vidx, vrows, sem):
        wid = lax.axis_index("s") * 2 + lax.axis_index("c")
        base = wid * B_PER_W
        pltpu.sync_copy(t_hbm.at[pl.ds(base, B_PER_W)], vidx)
        pltpu.async_copy(tab_hbm.at[vidx], vrows, sem).wait()
        pltpu.sync_copy(vrows, out_hbm.at[pl.ds(base, B_PER_W)])

    return k(jnp.asarray(_SCHED_TAB), t)


RS = 256


def _tc_body(ab_ref, m1_ref, cd_ref, nz_ref, out_ref):
    out_ref[...] = ab_ref[...] * cd_ref[...] + m1_ref[...] * nz_ref[...]


def kernel(clean_delta, t, noise):
    t = t.astype(jnp.int32)
    coeffs = _sc_gather_coeffs(t)
    ab = coeffs[:, 0].reshape(1, B)
    m1 = coeffs[:, 8].reshape(1, B)

    cd2 = clean_delta.transpose(1, 2, 0).reshape(ROW, B)
    nz2 = noise.transpose(1, 2, 0).reshape(ROW, B)

    out = pl.pallas_call(
        _tc_body,
        grid=(ROW // RS,),
        in_specs=[
            pl.BlockSpec((1, B), lambda i: (0, 0)),
            pl.BlockSpec((1, B), lambda i: (0, 0)),
            pl.BlockSpec((RS, B), lambda i: (i, 0)),
            pl.BlockSpec((RS, B), lambda i: (i, 0)),
        ],
        out_specs=pl.BlockSpec((RS, B), lambda i: (i, 0)),
        out_shape=jax.ShapeDtypeStruct((ROW, B), jnp.float32),
        compiler_params=pltpu.CompilerParams(
            dimension_semantics=("arbitrary",),
        ),
    )(ab, m1, cd2, nz2)
    return out.reshape(200, 64, B).transpose(2, 0, 1)


# speedup vs baseline: 5.8584x; 1.0012x over previous
import functools

import jax
import jax.numpy as jnp
import numpy as np
from jax import lax
from jax.experimental import pallas as pl
from jax.experimental.pallas import tpu as pltpu
from jax.experimental.pallas import tpu_sc as plsc

T_STEPS = 1000
BETA_START = 0.0001
BETA_END = 0.02
TAB_PAD = 1024

B = 4096
ROW = 200 * 64

NUM_WORKERS = 32
B_PER_W = B // NUM_WORKERS


TAB_W = 128


def _schedule_table():
    betas = np.linspace(BETA_START, BETA_END, T_STEPS).astype(np.float32)
    alphas = (np.float32(1.0) - betas).astype(np.float32)
    abars = np.cumprod(alphas, dtype=np.float32)
    abars = np.concatenate([np.ones((1,), np.float32), abars])
    sqrt_ab = np.sqrt(abars).astype(np.float32)
    sqrt_1m = np.sqrt(np.maximum(np.float32(1.0) - abars, np.float32(0.0)))
    sqrt_1m = sqrt_1m.astype(np.float32)
    tab = np.zeros((TAB_PAD, TAB_W), np.float32)
    tab[: abars.shape[0], :8] = sqrt_ab[:, None]
    tab[: abars.shape[0], 8:] = sqrt_1m[:, None]
    return tab


_SCHED_TAB = _schedule_table()


def _sc_gather_coeffs(t):
    mesh = plsc.VectorSubcoreMesh(core_axis_name="c", subcore_axis_name="s")

    @functools.partial(
        pl.kernel,
        mesh=mesh,
        out_type=jax.ShapeDtypeStruct((B, TAB_W), jnp.float32),
        scratch_types=[
            pltpu.VMEM((B_PER_W,), jnp.int32),
            pltpu.VMEM((B_PER_W, TAB_W), jnp.float32),
            pltpu.SemaphoreType.DMA,
        ],
    )
    def k(tab_hbm, t_hbm, out_hbm, vidx, vrows, sem):
        wid = lax.axis_index("s") * 2 + lax.axis_index("c")
        base = wid * B_PER_W
        pltpu.sync_copy(t_hbm.at[pl.ds(base, B_PER_W)], vidx)
        pltpu.async_copy(tab_hbm.at[vidx], vrows, sem).wait()
        pltpu.sync_copy(vrows, out_hbm.at[pl.ds(base, B_PER_W)])

    return k(jnp.asarray(_SCHED_TAB), t)


RS = 512


def _tc_body(ab_ref, m1_ref, cd_ref, nz_ref, out_ref):
    out_ref[...] = ab_ref[...] * cd_ref[...] + m1_ref[...] * nz_ref[...]


def kernel(clean_delta, t, noise):
    t = t.astype(jnp.int32)
    coeffs = _sc_gather_coeffs(t)
    ab = coeffs[:, 0].reshape(1, B)
    m1 = coeffs[:, 8].reshape(1, B)

    cd2 = clean_delta.transpose(1, 2, 0).reshape(ROW, B)
    nz2 = noise.transpose(1, 2, 0).reshape(ROW, B)

    out = pl.pallas_call(
        _tc_body,
        grid=(ROW // RS,),
        in_specs=[
            pl.BlockSpec((1, B), lambda i: (0, 0)),
            pl.BlockSpec((1, B), lambda i: (0, 0)),
            pl.BlockSpec((RS, B), lambda i: (i, 0)),
            pl.BlockSpec((RS, B), lambda i: (i, 0)),
        ],
        out_specs=pl.BlockSpec((RS, B), lambda i: (i, 0)),
        out_shape=jax.ShapeDtypeStruct((ROW, B), jnp.float32),
        compiler_params=pltpu.CompilerParams(
            dimension_semantics=("arbitrary",),
        ),
    )(ab, m1, cd2, nz2)
    return out.reshape(200, 64, B).transpose(2, 0, 1)
